# Initial kernel scaffold; baseline (speedup 1.0000x reference)
import jax
import jax.numpy as jnp
from jax.experimental import pallas as pl


def kernel(x, X, Y):
    raise NotImplementedError("write your pallas kernel here")


# speedup vs baseline: 2.3356x; 2.3356x over previous
import functools

import jax
import jax.numpy as jnp
from jax import lax
from jax.experimental import pallas as pl
from jax.experimental.pallas import tpu as pltpu
from jax.experimental.pallas import tpu_sc as plsc

_DEPTH = 14
_LANES = 16


def _fff_call(x, X, Y):
    B, D = x.shape
    NC, NS = 2, 16
    NW = NC * NS
    T = _LANES
    b_per_w = B // NW
    n_chunks = b_per_w // T
    n_dot = D // _LANES

    mesh = plsc.VectorSubcoreMesh(core_axis_name="c", subcore_axis_name="s")

    @functools.partial(
        pl.kernel,
        mesh=mesh,
        out_type=jax.ShapeDtypeStruct((B, D), jnp.float32),
        compiler_params=pltpu.CompilerParams(needs_layout_passes=False),
        scratch_types=[
            pltpu.VMEM((T, D), jnp.float32),
            pltpu.VMEM((T, D), jnp.float32),
            pltpu.VMEM((T, D), jnp.float32),
            pltpu.VMEM((T, D), jnp.float32),
            pltpu.VMEM((T,), jnp.int32),
            pltpu.VMEM((T,), jnp.float32),
            pltpu.SemaphoreType.DMA,
            pltpu.SemaphoreType.DMA,
        ],
    )
    def fff(x_hbm, X_hbm, Y_hbm, out_hbm,
            x_v, y_v, xg_v, yg_v, idx_v, lam_v, semx, semy):
        wid = lax.axis_index("s") * NC + lax.axis_index("c")
        base = wid * b_per_w

        lane = lax.iota(jnp.int32, T)

        def dot_pass(t, lam_vec):
            accs = [None] * 4
            for k in range(n_dot):
                sl = pl.ds(k * _LANES, _LANES)
                prod = x_v[t, sl] * xg_v[t, sl]
                a = k % 4
                accs[a] = prod if accs[a] is None else accs[a] + prod
            acc = (accs[0] + accs[1]) + (accs[2] + accs[3])
            return jnp.where(lane == t, jnp.sum(acc), lam_vec)

        def make_axpy(first):
            def axpy_pass(t, carry):
                lam_t = jnp.sum(jnp.where(lane == t, lam_v[...], 0.0))
                for k in range(n_dot):
                    sl = pl.ds(k * _LANES, _LANES)
                    if first:
                        y_v[t, sl] = lam_t * yg_v[t, sl]
                    else:
                        y_v[t, sl] = y_v[t, sl] + lam_t * yg_v[t, sl]
                return carry
            return axpy_pass

        def level(first):
            cx = pltpu.async_copy(X_hbm.at[idx_v], xg_v, semx)
            cy = pltpu.async_copy(Y_hbm.at[idx_v], yg_v, semy)
            cx.wait()
            lam_vec = lax.fori_loop(0, T, dot_pass,
                                    jnp.zeros((T,), jnp.float32))
            lam_v[...] = lam_vec
            cy.wait()
            lax.fori_loop(0, T, make_axpy(first), 0)
            node = idx_v[...]
            idx_v[...] = node * 2 + 1 + (lam_vec > 0.0).astype(jnp.int32)

        def chunk_body(ci, carry):
            tok0 = base + ci * T
            pltpu.sync_copy(x_hbm.at[pl.ds(tok0, T)], x_v)
            idx_v[...] = jnp.zeros((T,), jnp.int32)
            level(True)

            def lvl_body(_d, c):
                level(False)
                return c

            lax.fori_loop(1, _DEPTH, lvl_body, 0)
            pltpu.sync_copy(y_v, out_hbm.at[pl.ds(tok0, T)])
            return carry

        lax.fori_loop(0, n_chunks, chunk_body, 0)

    return fff(x, X, Y)


def kernel(x, X, Y):
    return _fff_call(x, X, Y)
